# baseline (speedup 1.0000x reference)
import functools

import jax
import jax.numpy as jnp
from jax import lax
from jax.experimental import pallas as pl
from jax.experimental.pallas import tpu as pltpu
from jax.experimental.pallas import tpu_sc as plsc

_NC, _NS = 2, 16
_NW = _NC * _NS
_R = 128
_G = 50
_NBUF = 2


def _sc_head_call(h, gidx, w, b16, N, D):
  per_w = N // _NW
  nchunk = per_w // _R
  assert nchunk % _NBUF == 0
  mesh = plsc.VectorSubcoreMesh(core_axis_name="c", subcore_axis_name="s")

  @functools.partial(
      pl.kernel,
      out_type=jax.ShapeDtypeStruct((N // _G, _G), jnp.float32),
      mesh=mesh,
      scratch_types=[
          pltpu.VMEM((per_w,), jnp.int32),
          pltpu.VMEM((_NBUF, _R, D), jnp.float32),
          pltpu.VMEM((per_w // _G, _G), jnp.float32),
          pltpu.VMEM((D,), jnp.float32),
          pltpu.VMEM((16,), jnp.float32),
          [pltpu.SemaphoreType.DMA] * _NBUF,
      ],
      compiler_params=pltpu.CompilerParams(needs_layout_passes=False),
  )
  def sc_head(h_hbm, idx_hbm, w_hbm, b_hbm, out_hbm,
              idx_v, rows_v, out_v, w_v, b_v, sems):
    wid = lax.axis_index("s") * _NC + lax.axis_index("c")
    base = wid * per_w
    pltpu.sync_copy(w_hbm, w_v)
    pltpu.sync_copy(b_hbm, b_v)
    pltpu.sync_copy(idx_hbm.at[pl.ds(base, per_w)], idx_v)
    ws = [w_v[pl.ds(16 * j, 16)] for j in range(D // 16)]
    bias_vec = b_v[pl.ds(0, 16)]
    lane = lax.iota(jnp.int32, 16)

    def start_gather(c, buf):
      pltpu.async_copy(h_hbm.at[idx_v.at[pl.ds(c * _R, _R)]], rows_v.at[buf],
                       sems[buf])

    def wait_gather(c, buf):
      pltpu.make_async_copy(h_hbm.at[idx_v.at[pl.ds(c * _R, _R)]],
                            rows_v.at[buf], sems[buf]).wait()

    def compute(c, buf):
      @pl.loop(0, _R // 16)
      def group_loop(rg):
        vec = bias_vec
        for l in range(16):
          r = rg * 16 + l
          acc = rows_v[buf, r, pl.ds(0, 16)] * ws[0]
          for j in range(1, D // 16):
            acc = acc + rows_v[buf, r, pl.ds(16 * j, 16)] * ws[j]
          vec = jnp.where(lane == l, vec + jnp.sum(acc), vec)
        p = c * _R + rg * 16 + lane
        plsc.store_scatter(out_v, [p // _G, lax.rem(p, _G)], vec)

    for b in range(_NBUF):
      start_gather(b, b)

    @pl.loop(0, nchunk, step=_NBUF)
    def ring_loop(g):
      for b in range(_NBUF):
        c = g + b
        wait_gather(c, b)
        compute(c, b)
        start_gather(jnp.minimum(c + _NBUF, nchunk - 1), b)

    for b in range(_NBUF):
      wait_gather(0, b)

    pltpu.sync_copy(out_v, out_hbm.at[pl.ds(wid * (per_w // _G), per_w // _G)])

  return sc_head(h, gidx, w, b16)


def kernel(hidden_states, chosen_padded, chosen_valid, W, b):
  B, S, D = hidden_states.shape
  G = chosen_padded.shape[1]
  N = B * G

  h = hidden_states.reshape(B * S, D)
  gidx = (jnp.arange(B, dtype=jnp.int32)[:, None] * S
          + chosen_padded.astype(jnp.int32)).reshape(N)
  w = W.reshape(D).astype(jnp.float32)
  b16 = jnp.broadcast_to(b.astype(jnp.float32), (16,))

  return _sc_head_call(h, gidx, w, b16, N, D)


# speedup vs baseline: 1.2069x; 1.2069x over previous
import functools

import jax
import jax.numpy as jnp
from jax import lax
from jax.experimental import pallas as pl
from jax.experimental.pallas import tpu as pltpu
from jax.experimental.pallas import tpu_sc as plsc

_NC, _NS = 2, 16
_NW = _NC * _NS
_R = 128
_G = 50
_NBUF = 5


def _sc_head_call(h, gidx, w, b16, N, D):
  per_w = N // _NW
  nchunk = per_w // _R
  assert nchunk % _NBUF == 0
  mesh = plsc.VectorSubcoreMesh(core_axis_name="c", subcore_axis_name="s")

  @functools.partial(
      pl.kernel,
      out_type=jax.ShapeDtypeStruct((N // _G, _G), jnp.float32),
      mesh=mesh,
      scratch_types=[
          pltpu.VMEM((per_w,), jnp.int32),
          pltpu.VMEM((_NBUF, _R, D), jnp.float32),
          pltpu.VMEM((per_w // _G, _G), jnp.float32),
          pltpu.VMEM((D,), jnp.float32),
          pltpu.VMEM((16,), jnp.float32),
          [pltpu.SemaphoreType.DMA] * _NBUF,
      ],
      compiler_params=pltpu.CompilerParams(needs_layout_passes=False),
  )
  def sc_head(h_hbm, idx_hbm, w_hbm, b_hbm, out_hbm,
              idx_v, rows_v, out_v, w_v, b_v, sems):
    wid = lax.axis_index("s") * _NC + lax.axis_index("c")
    base = wid * per_w
    pltpu.sync_copy(w_hbm, w_v)
    pltpu.sync_copy(b_hbm, b_v)
    pltpu.sync_copy(idx_hbm.at[pl.ds(base, per_w)], idx_v)
    ws = [w_v[pl.ds(16 * j, 16)] for j in range(D // 16)]
    bias_vec = b_v[pl.ds(0, 16)]
    lane = lax.iota(jnp.int32, 16)

    def start_gather(c, buf):
      pltpu.async_copy(h_hbm.at[idx_v.at[pl.ds(c * _R, _R)]], rows_v.at[buf],
                       sems[buf])

    def wait_gather(c, buf):
      pltpu.make_async_copy(h_hbm.at[idx_v.at[pl.ds(c * _R, _R)]],
                            rows_v.at[buf], sems[buf]).wait()

    def compute(c, buf):
      @pl.loop(0, _R // 16)
      def group_loop(rg):
        vec = bias_vec
        for l in range(16):
          r = rg * 16 + l
          acc = rows_v[buf, r, pl.ds(0, 16)] * ws[0]
          for j in range(1, D // 16):
            acc = acc + rows_v[buf, r, pl.ds(16 * j, 16)] * ws[j]
          vec = jnp.where(lane == l, vec + jnp.sum(acc), vec)
        p = c * _R + rg * 16 + lane
        plsc.store_scatter(out_v, [p // _G, lax.rem(p, _G)], vec)

    for b in range(_NBUF):
      start_gather(b, b)

    @pl.loop(0, nchunk, step=_NBUF)
    def ring_loop(g):
      for b in range(_NBUF):
        c = g + b
        wait_gather(c, b)
        compute(c, b)
        start_gather(jnp.minimum(c + _NBUF, nchunk - 1), b)

    for b in range(_NBUF):
      wait_gather(0, b)

    pltpu.sync_copy(out_v, out_hbm.at[pl.ds(wid * (per_w // _G), per_w // _G)])

  return sc_head(h, gidx, w, b16)


def kernel(hidden_states, chosen_padded, chosen_valid, W, b):
  B, S, D = hidden_states.shape
  G = chosen_padded.shape[1]
  N = B * G

  h = hidden_states.reshape(B * S, D)
  gidx = (jnp.arange(B, dtype=jnp.int32)[:, None] * S
          + chosen_padded.astype(jnp.int32)).reshape(N)
  w = W.reshape(D).astype(jnp.float32)
  b16 = jnp.broadcast_to(b.astype(jnp.float32), (16,))

  return _sc_head_call(h, gidx, w, b16, N, D)
